# Initial kernel scaffold; baseline (speedup 1.0000x reference)
import jax
import jax.numpy as jnp
from jax.experimental import pallas as pl


def kernel(x, xe, sgs, cf, W1, b1, a1, W2, b2, a2, Wa, va, Wf, bf, Wf2, bf2, Wf3, bf3, pair):
    raise NotImplementedError("write your pallas kernel here")


# speedup vs baseline: 1.0017x; 1.0017x over previous
import functools

import jax
import jax.numpy as jnp
from jax import lax
from jax.experimental import pallas as pl
from jax.experimental.pallas import tpu as pltpu

N = 10000
E = 5000
NNZ = 320000
D = 128
NH = 128
B = 1024
DCF = 16
NC = 10
LH = 2 * NH // 3

NPAD = 10240
POOL_BLK = 1024
POOL_STEPS = NPAD // POOL_BLK

_PREC = jax.lax.Precision.HIGHEST


def _seg_softmax(logits, seg, num):
    m = jax.ops.segment_max(logits, seg, num_segments=num)
    w = jnp.exp(logits - m[seg])
    s = jax.ops.segment_sum(w, seg, num_segments=num)
    return w / (s[seg] + 1e-9)


def _hgat_jax(x, xe, pair, a, W, b):
    x = x @ W + b
    xe = xe @ W + b
    ei, ni = pair[0], pair[1]
    pe = jax.nn.leaky_relu((xe[ei] * x[ni]) @ a, negative_slope=0.2)[:, 0]
    att_e = _seg_softmax(pe, ei, xe.shape[0])
    xe_o = jax.ops.segment_sum(att_e[:, None] * x[ni], ei, num_segments=xe.shape[0])
    att_n = _seg_softmax(pe, ni, x.shape[0])
    x_o = jax.ops.segment_sum(att_n[:, None] * xe_o[ei], ni, num_segments=x.shape[0])
    return x_o, xe_o


def _pool_head_body(x2_ref, sgs_ref, cf_ref, Wa_ref, va_ref, Wf_ref, bf_ref,
                    Wf2_ref, bf2_ref, Wf3_ref, bf3_ref,
                    out_ref, xsg_ref,
                    num_acc, den_acc, col_acc):
    j = pl.program_id(0)

    @pl.when(j == 0)
    def _init():
        num_acc[...] = jnp.zeros_like(num_acc)
        den_acc[...] = jnp.zeros_like(den_acc)
        col_acc[...] = jnp.zeros_like(col_acc)

    x2b = x2_ref[...]
    sgsb = sgs_ref[...]
    sb = jnp.dot(jnp.tanh(jnp.dot(x2b, Wa_ref[...], precision=_PREC,
                                  preferred_element_type=jnp.float32)),
                 va_ref[...], precision=_PREC,
                 preferred_element_type=jnp.float32)
    es = jnp.exp(sb)
    y2 = x2b * es
    num_acc[...] += jnp.dot(sgsb, y2, precision=_PREC,
                            preferred_element_type=jnp.float32)
    den_acc[...] += jnp.dot(sgsb, es, precision=_PREC,
                            preferred_element_type=jnp.float32)
    col_acc[...] += jnp.sum(x2b, axis=0, keepdims=True)

    @pl.when(j == POOL_STEPS - 1)
    def _final():
        den = den_acc[...]
        mean = col_acc[...] / N
        xsg = jnp.where(den > 0, num_acc[...] / jnp.where(den > 0, den, 1.0),
                        mean)
        xsg_ref[...] = xsg
        hcat = jnp.concatenate([xsg, cf_ref[...]], axis=1)
        h = jnp.maximum(jnp.dot(hcat, Wf_ref[...], precision=_PREC,
                                preferred_element_type=jnp.float32)
                        + bf_ref[...], 0.0)
        h = jnp.maximum(jnp.dot(h, Wf2_ref[...], precision=_PREC,
                                preferred_element_type=jnp.float32)
                        + bf2_ref[...], 0.0)
        out_ref[...] = jnp.dot(h, Wf3_ref[...], precision=_PREC,
                               preferred_element_type=jnp.float32) + bf3_ref[...]


def _pool_head(x2, sgs, cf, Wa, va, Wf, bf, Wf2, bf2, Wf3, bf3):
    x2p = jnp.zeros((NPAD, D), jnp.float32).at[:N].set(x2)
    sgsp = jnp.zeros((B, NPAD), jnp.float32).at[:, :N].set(sgs)
    full = lambda shape: pl.BlockSpec(shape, lambda j: (0,) * len(shape))
    out, xsg = pl.pallas_call(
        _pool_head_body,
        grid=(POOL_STEPS,),
        in_specs=[
            pl.BlockSpec((POOL_BLK, D), lambda j: (j, 0)),
            pl.BlockSpec((B, POOL_BLK), lambda j: (0, j)),
            full((B, DCF)),
            full((NH, NH)),
            full((NH, 1)),
            full((NH + DCF, LH)),
            full((LH,)),
            full((LH, LH)),
            full((LH,)),
            full((LH, NC)),
            full((NC,)),
        ],
        out_specs=[
            pl.BlockSpec((B, NC), lambda j: (0, 0)),
            pl.BlockSpec((B, NH), lambda j: (0, 0)),
        ],
        out_shape=[
            jax.ShapeDtypeStruct((B, NC), jnp.float32),
            jax.ShapeDtypeStruct((B, NH), jnp.float32),
        ],
        scratch_shapes=[
            pltpu.VMEM((B, NH), jnp.float32),
            pltpu.VMEM((B, 1), jnp.float32),
            pltpu.VMEM((1, D), jnp.float32),
        ],
    )(x2p, sgsp, cf, Wa, va, Wf, bf, Wf2, bf2, Wf3, bf3)
    return out, xsg


def kernel(x, xe, sgs, cf, W1, b1, a1, W2, b2, a2, Wa, va, Wf, bf, Wf2, bf2,
           Wf3, bf3, pair):
    x1, xe1 = _hgat_jax(x, xe, pair, a1, W1, b1)
    x2, xe2 = _hgat_jax(x1, xe1, pair, a2, W2, b2)
    out, xsg = _pool_head(x2, sgs, cf, Wa, va, Wf, bf, Wf2, bf2, Wf3, bf3)
    return (out, xsg, out, xe2)


# speedup vs baseline: 4.2717x; 4.2646x over previous
import dataclasses
import functools

import jax
import jax.numpy as jnp
from jax import lax
from jax.experimental import pallas as pl
from jax.experimental.pallas import tpu as pltpu
from jax.experimental.pallas import tpu_sc as plsc

N = 10000
E = 5000
NNZ = 320000
D = 128
NH = 128
B = 1024
DCF = 16
NCLS = 10
LH = 2 * NH // 3

EPAD = 5120
NPAD = 10240
CH = 128
NW = 32
NNZP = 327680
CHUNKS = NNZP // CH
CPT = CHUNKS // NW
NHALF = NPAD // 2
XUP = NHALF + 128
XTR = XUP // 16

POOL_BLK = 1024
POOL_STEPS = NPAD // POOL_BLK

_PREC = jax.lax.Precision.HIGHEST
_MESH = plsc.VectorSubcoreMesh(core_axis_name="c", subcore_axis_name="s")

_SC_PARAMS = pltpu.CompilerParams()
if "needs_layout_passes" in pltpu.CompilerParams.__dataclass_fields__:
    _SC_PARAMS = dataclasses.replace(_SC_PARAMS, needs_layout_passes=False)


@functools.partial(
    pl.kernel,
    out_type=[
        jax.ShapeDtypeStruct((2, EPAD, D), jnp.float32),
        jax.ShapeDtypeStruct((NW, EPAD), jnp.float32),
        jax.ShapeDtypeStruct((NW, NPAD), jnp.float32),
        jax.ShapeDtypeStruct((NNZP, 16), jnp.float32),
    ],
    mesh=_MESH,
    compiler_params=_SC_PARAMS,
    scratch_types=[
        pltpu.VMEM((CH,), jnp.int32),
        pltpu.VMEM((CH,), jnp.int32),
        pltpu.VMEM((CH, D), jnp.float32),
        pltpu.VMEM((CH, D), jnp.float32),
        pltpu.VMEM((CH, D), jnp.float32),
        pltpu.VMEM((CH, 16), jnp.float32),
        pltpu.VMEM((EPAD,), jnp.float32),
        pltpu.VMEM((NPAD,), jnp.float32),
        pltpu.VMEM_SHARED((EPAD, D), jnp.float32),
        pltpu.SemaphoreType.DMA,
        pltpu.SemaphoreType.DMA,
    ],
)
def _sc_pass1(uep_hbm, xp_hbm, ei_hbm, ni_hbm, zacc_hbm, z1d_hbm,
              acc_out, se_out, sn_out, w_out,
              ei_v, ni_v, ue_v, xr_v, val_v, w_v, se_t, sn_t,
              acc_sh, sem1, sem2):
    c = lax.axis_index("c")
    s = lax.axis_index("s")
    wid = s * 2 + c

    pltpu.sync_copy(zacc_hbm, acc_sh.at[pl.ds(s * (EPAD // 16), EPAD // 16)])
    pltpu.sync_copy(z1d_hbm.at[pl.ds(0, EPAD)], se_t)
    pltpu.sync_copy(z1d_hbm, sn_t)
    plsc.subcore_barrier()

    lane0 = lax.iota(jnp.int32, 16) == 0

    @pl.loop(0, CPT)
    def _chunks(t):
        base = (t * NW + wid) * CH
        pltpu.sync_copy(ei_hbm.at[pl.ds(base, CH)], ei_v)
        pltpu.sync_copy(ni_hbm.at[pl.ds(base, CH)], ni_v)
        g1 = pltpu.async_copy(uep_hbm.at[ei_v], ue_v, sem1)
        g2 = pltpu.async_copy(xp_hbm.at[ni_v], xr_v, sem2)
        g1.wait()
        g2.wait()

        @pl.loop(0, CH // 16)
        def _groups(g):
            ev16 = ei_v[pl.ds(g * 16, 16)]
            nv16 = ni_v[pl.ds(g * 16, 16)]
            for i in range(16):
                p = g * 16 + i
                acc = ue_v[p, pl.ds(0, 16)] * xr_v[p, pl.ds(0, 16)]
                for j in range(1, 8):
                    acc = acc + ue_v[p, pl.ds(16 * j, 16)] * xr_v[p, pl.ds(16 * j, 16)]
                pe = jnp.sum(acc)
                pe = jnp.where(pe >= 0.0, pe, 0.2 * pe)
                wv = jnp.exp(jnp.full((16,), pe, jnp.float32))
                w_v[p, pl.ds(0, 16)] = wv
                for j in range(8):
                    val_v[p, pl.ds(16 * j, 16)] = wv * xr_v[p, pl.ds(16 * j, 16)]
                eidx = jnp.full((16,), ev16[i], jnp.int32)
                nidx = jnp.full((16,), nv16[i], jnp.int32)
                plsc.addupdate_scatter(se_t, [eidx], wv, mask=lane0)
                plsc.addupdate_scatter(sn_t, [nidx], wv, mask=lane0)

        pltpu.sync_copy(val_v, acc_sh.at[ei_v], add=True)
        pltpu.sync_copy(w_v, w_out.at[pl.ds(base, CH)])

    plsc.subcore_barrier()
    eslc = pl.ds(s * (EPAD // 16), EPAD // 16)
    pltpu.sync_copy(acc_sh.at[eslc], acc_out.at[c, eslc])
    pltpu.sync_copy(se_t, se_out.at[wid])
    pltpu.sync_copy(sn_t, sn_out.at[wid])


@functools.partial(
    pl.kernel,
    out_type=jax.ShapeDtypeStruct((2, XUP, D), jnp.float32),
    mesh=_MESH,
    compiler_params=_SC_PARAMS,
    scratch_types=[
        pltpu.VMEM((CH,), jnp.int32),
        pltpu.VMEM((CH,), jnp.int32),
        pltpu.VMEM((CH,), jnp.int32),
        pltpu.VMEM((CH, D), jnp.float32),
        pltpu.VMEM((CH, D), jnp.float32),
        pltpu.VMEM((CH, 16), jnp.float32),
        pltpu.VMEM_SHARED((XUP, D), jnp.float32),
        pltpu.SemaphoreType.DMA,
    ],
)
def _sc_pass2(xeo_hbm, ei_hbm, ni_hbm, w_hbm, zxu_hbm, xu_out,
              ei_v, ni_v, ni2_v, xe_v, xval_v, w_v, xu_sh, sem1):
    c = lax.axis_index("c")
    s = lax.axis_index("s")
    offs = c * NHALF

    pltpu.sync_copy(zxu_hbm, xu_sh.at[pl.ds(s * XTR, XTR)])
    plsc.subcore_barrier()

    trash = jnp.full((16,), NHALF, jnp.int32)

    @pl.loop(0, CHUNKS // 16)
    def _chunks(t):
        base = (t * 16 + s) * CH
        pltpu.sync_copy(ei_hbm.at[pl.ds(base, CH)], ei_v)
        pltpu.sync_copy(ni_hbm.at[pl.ds(base, CH)], ni_v)
        pltpu.sync_copy(w_hbm.at[pl.ds(base, CH)], w_v)
        pltpu.async_copy(xeo_hbm.at[ei_v], xe_v, sem1).wait()

        @pl.loop(0, CH // 16)
        def _groups(g):
            nv16 = ni_v[pl.ds(g * 16, 16)]
            lidx = nv16 - offs
            ok = (lidx >= 0) & (lidx < NHALF)
            ni2_v[pl.ds(g * 16, 16)] = jnp.where(ok, lidx, trash)
            for i in range(16):
                p = g * 16 + i
                wv = w_v[p, pl.ds(0, 16)]
                for j in range(8):
                    xval_v[p, pl.ds(16 * j, 16)] = wv * xe_v[p, pl.ds(16 * j, 16)]

        pltpu.sync_copy(xval_v, xu_sh.at[ni2_v], add=True)

    plsc.subcore_barrier()
    nslc = pl.ds(s * XTR, XTR)
    pltpu.sync_copy(xu_sh.at[nslc], xu_out.at[c, nslc])


def _lin_body(x_ref, w_ref, b_ref, o_ref):
    o_ref[...] = (jnp.dot(x_ref[...], w_ref[...], precision=_PREC,
                          preferred_element_type=jnp.float32) + b_ref[...])


def _lin(x, w, b):
    rows = x.shape[0]
    return pl.pallas_call(
        _lin_body,
        grid=(rows // 1024,),
        in_specs=[
            pl.BlockSpec((1024, D), lambda i: (i, 0)),
            pl.BlockSpec((D, NH), lambda i: (0, 0)),
            pl.BlockSpec((1, NH), lambda i: (0, 0)),
        ],
        out_specs=pl.BlockSpec((1024, NH), lambda i: (i, 0)),
        out_shape=jax.ShapeDtypeStruct((rows, NH), jnp.float32),
    )(x, w, b)


def _norm_e_body(x0_ref, x1_ref, s_ref, o_ref):
    ssum = jnp.sum(s_ref[...], axis=0)[:, None]
    o_ref[...] = (x0_ref[0] + x1_ref[0]) / (ssum + 1e-9)


def _norm_e(acc, ssum):
    return pl.pallas_call(
        _norm_e_body,
        grid=(EPAD // 1024,),
        in_specs=[
            pl.BlockSpec((1, 1024, D), lambda i: (0, i, 0)),
            pl.BlockSpec((1, 1024, D), lambda i: (1, i, 0)),
            pl.BlockSpec((NW, 1024), lambda i: (0, i)),
        ],
        out_specs=pl.BlockSpec((1024, D), lambda i: (i, 0)),
        out_shape=jax.ShapeDtypeStruct((EPAD, D), jnp.float32),
    )(acc, acc, ssum)


def _norm_n_body(x_ref, s_ref, o_ref):
    ssum = jnp.sum(s_ref[...], axis=0)[:, None]
    o_ref[...] = x_ref[0] / (ssum + 1e-9)


def _norm_n(xu, ssum):
    nblk = NHALF // 1024
    return pl.pallas_call(
        _norm_n_body,
        grid=(NPAD // 1024,),
        in_specs=[
            pl.BlockSpec((1, 1024, D), lambda i: (i // nblk, i % nblk, 0)),
            pl.BlockSpec((NW, 1024), lambda i: (0, i)),
        ],
        out_specs=pl.BlockSpec((1024, D), lambda i: (i, 0)),
        out_shape=jax.ShapeDtypeStruct((NPAD, D), jnp.float32),
    )(xu, ssum)


def _pool_head_body(x2_ref, sgs_ref, cf_ref, Wa_ref, va_ref, Wf_ref, bf_ref,
                    Wf2_ref, bf2_ref, Wf3_ref, bf3_ref,
                    out_ref, xsg_ref,
                    num_acc, den_acc, col_acc):
    j = pl.program_id(0)

    @pl.when(j == 0)
    def _init():
        num_acc[...] = jnp.zeros_like(num_acc)
        den_acc[...] = jnp.zeros_like(den_acc)
        col_acc[...] = jnp.zeros_like(col_acc)

    x2b = x2_ref[...]
    sgsb = sgs_ref[...]
    sb = jnp.dot(jnp.tanh(jnp.dot(x2b, Wa_ref[...], precision=_PREC,
                                  preferred_element_type=jnp.float32)),
                 va_ref[...], precision=_PREC,
                 preferred_element_type=jnp.float32)
    es = jnp.exp(sb)
    y2 = x2b * es
    num_acc[...] += jnp.dot(sgsb, y2, precision=_PREC,
                            preferred_element_type=jnp.float32)
    den_acc[...] += jnp.dot(sgsb, es, precision=_PREC,
                            preferred_element_type=jnp.float32)
    rowid = lax.broadcasted_iota(jnp.int32, (POOL_BLK, 1), 0) + j * POOL_BLK
    col_acc[...] += jnp.sum(jnp.where(rowid < N, x2b, 0.0), axis=0,
                            keepdims=True)

    @pl.when(j == POOL_STEPS - 1)
    def _final():
        den = den_acc[...]
        mean = col_acc[...] / N
        xsg = jnp.where(den > 0, num_acc[...] / jnp.where(den > 0, den, 1.0),
                        mean)
        xsg_ref[...] = xsg
        hcat = jnp.concatenate([xsg, cf_ref[...]], axis=1)
        h = jnp.maximum(jnp.dot(hcat, Wf_ref[...], precision=_PREC,
                                preferred_element_type=jnp.float32)
                        + bf_ref[...], 0.0)
        h = jnp.maximum(jnp.dot(h, Wf2_ref[...], precision=_PREC,
                                preferred_element_type=jnp.float32)
                        + bf2_ref[...], 0.0)
        out_ref[...] = jnp.dot(h, Wf3_ref[...], precision=_PREC,
                               preferred_element_type=jnp.float32) + bf3_ref[...]


def _pool_head(x2p, sgsp, cf, Wa, va, Wf, bf, Wf2, bf2, Wf3, bf3):
    full = lambda shape: pl.BlockSpec(shape, lambda j: (0,) * len(shape))
    out, xsg = pl.pallas_call(
        _pool_head_body,
        grid=(POOL_STEPS,),
        in_specs=[
            pl.BlockSpec((POOL_BLK, D), lambda j: (j, 0)),
            pl.BlockSpec((B, POOL_BLK), lambda j: (0, j)),
            full((B, DCF)),
            full((NH, NH)),
            full((NH, 1)),
            full((NH + DCF, LH)),
            full((LH,)),
            full((LH, LH)),
            full((LH,)),
            full((LH, NCLS)),
            full((NCLS,)),
        ],
        out_specs=[
            pl.BlockSpec((B, NCLS), lambda j: (0, 0)),
            pl.BlockSpec((B, NH), lambda j: (0, 0)),
        ],
        out_shape=[
            jax.ShapeDtypeStruct((B, NCLS), jnp.float32),
            jax.ShapeDtypeStruct((B, NH), jnp.float32),
        ],
        scratch_shapes=[
            pltpu.VMEM((B, NH), jnp.float32),
            pltpu.VMEM((B, 1), jnp.float32),
            pltpu.VMEM((1, D), jnp.float32),
        ],
    )(x2p, sgsp, cf, Wa, va, Wf, bf, Wf2, bf2, Wf3, bf3)
    return out, xsg


def kernel(x, xe, sgs, cf, W1, b1, a1, W2, b2, a2, Wa, va, Wf, bf, Wf2, bf2,
           Wf3, bf3, pair):
    f32 = jnp.float32
    xpad = jnp.zeros((NPAD, D), f32).at[:N].set(x)
    xepad = jnp.zeros((EPAD, D), f32).at[:E].set(xe)
    npad = NNZP - NNZ
    eip = jnp.concatenate([pair[0], jnp.full((npad,), EPAD - 1, jnp.int32)])
    nip = jnp.concatenate([pair[1], jnp.full((npad,), NPAD - 1, jnp.int32)])
    sgsp = jnp.zeros((B, NPAD), f32).at[:, :N].set(sgs)
    zacc = jnp.zeros((EPAD // 16, D), f32)
    z1d = jnp.zeros((NPAD,), f32)
    zxu = jnp.zeros((XTR, D), f32)

    def layer(xin, xein, W, b, a):
        xp = _lin(xin, W, b.reshape(1, NH))
        uep = _lin(xein, W * a[:, 0][None, :], (b * a[:, 0]).reshape(1, NH))
        acc, se, sn, w = _sc_pass1(uep, xp, eip, nip, zacc, z1d)
        xeo = _norm_e(acc, se)
        xu = _sc_pass2(xeo, eip, nip, w, zxu)
        xo = _norm_n(xu, sn)
        return xo, xeo

    x1, xe1 = layer(xpad, xepad, W1, b1, a1)
    x2, xe2p = layer(x1, xe1, W2, b2, a2)
    out, xsg = _pool_head(x2, sgsp, cf, Wa, va, Wf, bf, Wf2, bf2, Wf3, bf3)
    return (out, xsg, out, xe2p[:E])


# speedup vs baseline: 5.3117x; 1.2434x over previous
import dataclasses
import functools

import jax
import jax.numpy as jnp
from jax import lax
from jax.experimental import pallas as pl
from jax.experimental.pallas import tpu as pltpu
from jax.experimental.pallas import tpu_sc as plsc

N = 10000
E = 5000
NNZ = 320000
D = 128
NH = 128
B = 1024
DCF = 16
NCLS = 10
LH = 2 * NH // 3

EPAD = 5120
NPAD = 10240
CH = 128
CH1 = 64
NW = 32
NNZP = 327680
CHUNKS = NNZP // CH
CHUNKS1 = NNZP // CH1
CPT = CHUNKS1 // NW
NHALF = NPAD // 2
XUP = NHALF + 128
XTR = XUP // 16
EACC = 5120
EDUM = EACC - 1

POOL_BLK = 1024
POOL_STEPS = NPAD // POOL_BLK

_PREC = jax.lax.Precision.HIGHEST
_MESH = plsc.VectorSubcoreMesh(core_axis_name="c", subcore_axis_name="s")

_SC_PARAMS = pltpu.CompilerParams()
if "needs_layout_passes" in pltpu.CompilerParams.__dataclass_fields__:
    _SC_PARAMS = dataclasses.replace(_SC_PARAMS, needs_layout_passes=False)


@functools.partial(
    pl.kernel,
    out_type=[
        jax.ShapeDtypeStruct((2, EACC, D), jnp.float32),
        jax.ShapeDtypeStruct((NW, EPAD), jnp.float32),
        jax.ShapeDtypeStruct((NW, NPAD), jnp.float32),
        jax.ShapeDtypeStruct((NNZP, 16), jnp.float32),
    ],
    mesh=_MESH,
    compiler_params=_SC_PARAMS,
    scratch_types=[
        pltpu.VMEM((CH1,), jnp.int32),
        pltpu.VMEM((CH1,), jnp.int32),
        pltpu.VMEM((CH1,), jnp.int32),
        pltpu.VMEM((CH1,), jnp.int32),
        pltpu.VMEM((CH1, D), jnp.float32),
        pltpu.VMEM((CH1, D), jnp.float32),
        pltpu.VMEM((CH1, D), jnp.float32),
        pltpu.VMEM((CH1, D), jnp.float32),
        pltpu.VMEM((CH1, D), jnp.float32),
        pltpu.VMEM((CH1, 16), jnp.float32),
        pltpu.VMEM((EPAD,), jnp.float32),
        pltpu.VMEM((NPAD,), jnp.float32),
        pltpu.VMEM_SHARED((EACC, D), jnp.float32),
        pltpu.SemaphoreType.DMA,
        pltpu.SemaphoreType.DMA,
    ],
)
def _sc_pass1(uep_hbm, xp_hbm, ei_hbm, ni_hbm, zacc_hbm, z1d_hbm,
              acc_out, se_out, sn_out, w_out,
              ei_a, ni_a, ei_b, ni_b, ue_a, xr_a, ue_b, xr_b, val_v,
              w_v, se_t, sn_t, acc_sh, sem_a, sem_b):
    c = lax.axis_index("c")
    s = lax.axis_index("s")
    wid = s * 2 + c
    first = wid * CPT
    last = first + CPT - 1

    pltpu.sync_copy(zacc_hbm, acc_sh.at[pl.ds(s * (EACC // 16), EACC // 16)])
    pltpu.sync_copy(z1d_hbm.at[pl.ds(0, EPAD)], se_t)
    pltpu.sync_copy(z1d_hbm, sn_t)
    plsc.subcore_barrier()

    lane0 = lax.iota(jnp.int32, 16) == 0

    def fetch(t, ei_v, ni_v, ue_v, xr_v, sem):
        pltpu.sync_copy(ei_hbm.at[pl.ds(t * CH1, CH1)], ei_v)
        pltpu.sync_copy(ni_hbm.at[pl.ds(t * CH1, CH1)], ni_v)
        g1 = pltpu.async_copy(uep_hbm.at[ei_v], ue_v, sem)
        g2 = pltpu.async_copy(xp_hbm.at[ni_v], xr_v, sem)
        return g1, g2

    def process(t, ei_v, ni_v, ue_v, xr_v):
        @pl.loop(0, CH1 // 16)
        def _groups(g):
            ev16 = ei_v[pl.ds(g * 16, 16)]
            nv16 = ni_v[pl.ds(g * 16, 16)]
            for i in range(16):
                p = g * 16 + i
                acc = ue_v[p, pl.ds(0, 16)] * xr_v[p, pl.ds(0, 16)]
                for j in range(1, 8):
                    acc = acc + ue_v[p, pl.ds(16 * j, 16)] * xr_v[p, pl.ds(16 * j, 16)]
                pe = jnp.sum(acc)
                pe = jnp.where(pe >= 0.0, pe, 0.2 * pe)
                wv = jnp.exp(jnp.full((16,), pe, jnp.float32))
                w_v[p, pl.ds(0, 16)] = wv
                for j in range(8):
                    val_v[p, pl.ds(16 * j, 16)] = wv * xr_v[p, pl.ds(16 * j, 16)]
                eidx = jnp.full((16,), ev16[i], jnp.int32)
                nidx = jnp.full((16,), nv16[i], jnp.int32)
                plsc.addupdate_scatter(se_t, [eidx], wv, mask=lane0)
                plsc.addupdate_scatter(sn_t, [nidx], wv, mask=lane0)

        pltpu.sync_copy(val_v, acc_sh.at[ei_v], add=True)
        pltpu.sync_copy(w_v, w_out.at[pl.ds(t * CH1, CH1)])

    ga = fetch(first, ei_a, ni_a, ue_a, xr_a, sem_a)

    @pl.loop(0, CPT // 2)
    def _chunks(u):
        t0 = first + 2 * u
        for g in ga:
            g.wait()
        gb = fetch(t0 + 1, ei_b, ni_b, ue_b, xr_b, sem_b)
        process(t0, ei_a, ni_a, ue_a, xr_a)
        for g in gb:
            g.wait()
        ga2 = fetch(jnp.minimum(t0 + 2, last), ei_a, ni_a, ue_a, xr_a, sem_a)
        process(t0 + 1, ei_b, ni_b, ue_b, xr_b)

    for g in ga:
        g.wait()

    plsc.subcore_barrier()

    eslc = pl.ds(s * (EACC // 16), EACC // 16)
    pltpu.sync_copy(acc_sh.at[eslc], acc_out.at[c, eslc])
    pltpu.sync_copy(se_t, se_out.at[wid])
    pltpu.sync_copy(sn_t, sn_out.at[wid])


CPT2 = CHUNKS // 16


@functools.partial(
    pl.kernel,
    out_type=jax.ShapeDtypeStruct((2, XUP, D), jnp.float32),
    mesh=_MESH,
    compiler_params=_SC_PARAMS,
    scratch_types=[
        pltpu.VMEM((CH,), jnp.int32),
        pltpu.VMEM((CH,), jnp.int32),
        pltpu.VMEM((CH,), jnp.int32),
        pltpu.VMEM((CH,), jnp.int32),
        pltpu.VMEM((CH,), jnp.int32),
        pltpu.VMEM((CH, D), jnp.float32),
        pltpu.VMEM((CH, D), jnp.float32),
        pltpu.VMEM((CH, D), jnp.float32),
        pltpu.VMEM((CH, 16), jnp.float32),
        pltpu.VMEM((CH, 16), jnp.float32),
        pltpu.VMEM_SHARED((XUP, D), jnp.float32),
        pltpu.SemaphoreType.DMA,
        pltpu.SemaphoreType.DMA,
    ],
)
def _sc_pass2(xeo_hbm, ei_hbm, ni_hbm, w_hbm, zxu_hbm, xu_out,
              ei_a, ni_a, ei_b, ni_b, ni2_v, xe_a, xe_b, xval_v, w_a, w_b,
              xu_sh, sem_a, sem_b):
    c = lax.axis_index("c")
    s = lax.axis_index("s")
    offs = c * NHALF
    first = s * CPT2
    last = first + CPT2 - 1

    pltpu.sync_copy(zxu_hbm, xu_sh.at[pl.ds(s * XTR, XTR)])
    plsc.subcore_barrier()

    trash = jnp.full((16,), NHALF, jnp.int32)

    def fetch(t, ei_v, ni_v, xe_v, w_v, sem):
        pltpu.sync_copy(ei_hbm.at[pl.ds(t * CH, CH)], ei_v)
        pltpu.sync_copy(ni_hbm.at[pl.ds(t * CH, CH)], ni_v)
        g1 = pltpu.async_copy(xeo_hbm.at[ei_v], xe_v, sem)
        g2 = pltpu.async_copy(w_hbm.at[pl.ds(t * CH, CH)], w_v, sem)
        return g1, g2

    def process(ni_v, xe_v, w_v):
        @pl.loop(0, CH // 16)
        def _groups(g):
            nv16 = ni_v[pl.ds(g * 16, 16)]
            lidx = nv16 - offs
            ok = (lidx >= 0) & (lidx < NHALF)
            ni2_v[pl.ds(g * 16, 16)] = jnp.where(ok, lidx, trash)
            for i in range(16):
                p = g * 16 + i
                wv = w_v[p, pl.ds(0, 16)]
                for j in range(8):
                    xval_v[p, pl.ds(16 * j, 16)] = wv * xe_v[p, pl.ds(16 * j, 16)]

        pltpu.sync_copy(xval_v, xu_sh.at[ni2_v], add=True)

    ga = fetch(first, ei_a, ni_a, xe_a, w_a, sem_a)

    @pl.loop(0, CPT2 // 2)
    def _chunks(u):
        t0 = first + 2 * u
        for g in ga:
            g.wait()
        gb = fetch(t0 + 1, ei_b, ni_b, xe_b, w_b, sem_b)
        process(ni_a, xe_a, w_a)
        for g in gb:
            g.wait()
        ga2 = fetch(jnp.minimum(t0 + 2, last), ei_a, ni_a, xe_a, w_a, sem_a)
        process(ni_b, xe_b, w_b)

    for g in ga:
        g.wait()

    plsc.subcore_barrier()

    nslc = pl.ds(s * XTR, XTR)
    pltpu.sync_copy(xu_sh.at[nslc], xu_out.at[c, nslc])


def _lin_body(x_ref, w_ref, b_ref, o_ref):
    o_ref[...] = (jnp.dot(x_ref[...], w_ref[...], precision=_PREC,
                          preferred_element_type=jnp.float32) + b_ref[...])


def _lin(x, w, b):
    rows = x.shape[0]
    return pl.pallas_call(
        _lin_body,
        grid=(rows // 1024,),
        in_specs=[
            pl.BlockSpec((1024, D), lambda i: (i, 0)),
            pl.BlockSpec((D, NH), lambda i: (0, 0)),
            pl.BlockSpec((1, NH), lambda i: (0, 0)),
        ],
        out_specs=pl.BlockSpec((1024, NH), lambda i: (i, 0)),
        out_shape=jax.ShapeDtypeStruct((rows, NH), jnp.float32),
    )(x, w, b)


def _norm_e_body(x0_ref, x1_ref, s_ref, o_ref):
    ssum = jnp.sum(s_ref[...], axis=0)[:, None]
    o_ref[...] = (x0_ref[0] + x1_ref[0]) / (ssum + 1e-9)


def _norm_e(acc, ssum):
    return pl.pallas_call(
        _norm_e_body,
        grid=((EACC + 1023) // 1024,),
        in_specs=[
            pl.BlockSpec((1, 1024, D), lambda i: (0, i, 0)),
            pl.BlockSpec((1, 1024, D), lambda i: (1, i, 0)),
            pl.BlockSpec((NW, 1024), lambda i: (0, i)),
        ],
        out_specs=pl.BlockSpec((1024, D), lambda i: (i, 0)),
        out_shape=jax.ShapeDtypeStruct((EACC, D), jnp.float32),
    )(acc, acc, ssum)


def _norm_n_body(x_ref, s_ref, o_ref):
    ssum = jnp.sum(s_ref[...], axis=0)[:, None]
    o_ref[...] = x_ref[0] / (ssum + 1e-9)


def _norm_n(xu, ssum):
    nblk = NHALF // 1024
    return pl.pallas_call(
        _norm_n_body,
        grid=(NPAD // 1024,),
        in_specs=[
            pl.BlockSpec((1, 1024, D), lambda i: (i // nblk, i % nblk, 0)),
            pl.BlockSpec((NW, 1024), lambda i: (0, i)),
        ],
        out_specs=pl.BlockSpec((1024, D), lambda i: (i, 0)),
        out_shape=jax.ShapeDtypeStruct((NPAD, D), jnp.float32),
    )(xu, ssum)


def _pool_head_body(x2_ref, sgs_ref, cf_ref, Wa_ref, va_ref, Wf_ref, bf_ref,
                    Wf2_ref, bf2_ref, Wf3_ref, bf3_ref,
                    out_ref, xsg_ref,
                    num_acc, den_acc, col_acc):
    j = pl.program_id(0)

    @pl.when(j == 0)
    def _init():
        num_acc[...] = jnp.zeros_like(num_acc)
        den_acc[...] = jnp.zeros_like(den_acc)
        col_acc[...] = jnp.zeros_like(col_acc)

    x2b = x2_ref[...]
    sgsb = sgs_ref[...]
    sb = jnp.dot(jnp.tanh(jnp.dot(x2b, Wa_ref[...], precision=_PREC,
                                  preferred_element_type=jnp.float32)),
                 va_ref[...], precision=_PREC,
                 preferred_element_type=jnp.float32)
    es = jnp.exp(sb)
    y2 = x2b * es
    num_acc[...] += jnp.dot(sgsb, y2, precision=_PREC,
                            preferred_element_type=jnp.float32)
    den_acc[...] += jnp.dot(sgsb, es, precision=_PREC,
                            preferred_element_type=jnp.float32)
    rowid = lax.broadcasted_iota(jnp.int32, (POOL_BLK, 1), 0) + j * POOL_BLK
    col_acc[...] += jnp.sum(jnp.where(rowid < N, x2b, 0.0), axis=0,
                            keepdims=True)

    @pl.when(j == POOL_STEPS - 1)
    def _final():
        den = den_acc[...]
        mean = col_acc[...] / N
        xsg = jnp.where(den > 0, num_acc[...] / jnp.where(den > 0, den, 1.0),
                        mean)
        xsg_ref[...] = xsg
        hcat = jnp.concatenate([xsg, cf_ref[...]], axis=1)
        h = jnp.maximum(jnp.dot(hcat, Wf_ref[...], precision=_PREC,
                                preferred_element_type=jnp.float32)
                        + bf_ref[...], 0.0)
        h = jnp.maximum(jnp.dot(h, Wf2_ref[...], precision=_PREC,
                                preferred_element_type=jnp.float32)
                        + bf2_ref[...], 0.0)
        out_ref[...] = jnp.dot(h, Wf3_ref[...], precision=_PREC,
                               preferred_element_type=jnp.float32) + bf3_ref[...]


def _pool_head(x2p, sgsp, cf, Wa, va, Wf, bf, Wf2, bf2, Wf3, bf3):
    full = lambda shape: pl.BlockSpec(shape, lambda j: (0,) * len(shape))
    out, xsg = pl.pallas_call(
        _pool_head_body,
        grid=(POOL_STEPS,),
        in_specs=[
            pl.BlockSpec((POOL_BLK, D), lambda j: (j, 0)),
            pl.BlockSpec((B, POOL_BLK), lambda j: (0, j)),
            full((B, DCF)),
            full((NH, NH)),
            full((NH, 1)),
            full((NH + DCF, LH)),
            full((LH,)),
            full((LH, LH)),
            full((LH,)),
            full((LH, NCLS)),
            full((NCLS,)),
        ],
        out_specs=[
            pl.BlockSpec((B, NCLS), lambda j: (0, 0)),
            pl.BlockSpec((B, NH), lambda j: (0, 0)),
        ],
        out_shape=[
            jax.ShapeDtypeStruct((B, NCLS), jnp.float32),
            jax.ShapeDtypeStruct((B, NH), jnp.float32),
        ],
        scratch_shapes=[
            pltpu.VMEM((B, NH), jnp.float32),
            pltpu.VMEM((B, 1), jnp.float32),
            pltpu.VMEM((1, D), jnp.float32),
        ],
    )(x2p, sgsp, cf, Wa, va, Wf, bf, Wf2, bf2, Wf3, bf3)
    return out, xsg


def kernel(x, xe, sgs, cf, W1, b1, a1, W2, b2, a2, Wa, va, Wf, bf, Wf2, bf2,
           Wf3, bf3, pair):
    f32 = jnp.float32
    xpad = jnp.zeros((NPAD, D), f32).at[:N].set(x)
    xepad = jnp.zeros((EPAD, D), f32).at[:E].set(xe)
    npad = NNZP - NNZ
    eip = jnp.concatenate([pair[0], jnp.full((npad,), EDUM, jnp.int32)])
    nip = jnp.concatenate([pair[1], jnp.full((npad,), NPAD - 1, jnp.int32)])
    sgsp = jnp.zeros((B, NPAD), f32).at[:, :N].set(sgs)
    zacc = jnp.zeros((EACC // 16, D), f32)
    z1d = jnp.zeros((NPAD,), f32)
    zxu = jnp.zeros((XTR, D), f32)

    def layer(xin, xein, W, b, a):
        xp = _lin(xin, W, b.reshape(1, NH))
        uep = _lin(xein, W * a[:, 0][None, :], (b * a[:, 0]).reshape(1, NH))
        acc, se, sn, w = _sc_pass1(uep, xp, eip, nip, zacc, z1d)
        xeo = _norm_e(acc, se)
        xu = _sc_pass2(xeo, eip, nip, w, zxu)
        xo = _norm_n(xu, sn)
        return xo, xeo

    x1, xe1 = layer(xpad, xepad, W1, b1, a1)
    x2, xe2p = layer(x1, xe1, W2, b2, a2)
    out, xsg = _pool_head(x2, sgsp, cf, Wa, va, Wf, bf, Wf2, bf2, Wf3, bf3)
    return (out, xsg, out, xe2p[:E])
